# Initial kernel scaffold; baseline (speedup 1.0000x reference)
import jax
import jax.numpy as jnp
from jax.experimental import pallas as pl


def kernel(E_pos, E_item, u, v, w, n, edge_index):
    raise NotImplementedError("write your pallas kernel here")


# speedup vs baseline: 10.4500x; 10.4500x over previous
import functools

import jax
import jax.numpy as jnp
from jax import lax
from jax.experimental import pallas as pl
from jax.experimental.pallas import tpu as pltpu
from jax.experimental.pallas import tpu_sc as plsc

NN = 50000
D = 64
HD = 32
E = 800000
B = 4096
NNEG = 40
REG = 1e-4

NP = 50176
EP = 819200
ER = EP // 128
DUMMY = NN

NC, NS = 2, 16
RPT = NP // NS
CR = 112
KB = 4
EPTR = ER // NS
NIT = EPTR // KB

GI = 2 * B + B * NNEG
GRP = 1408
GIP = GRP * 128
GPT = GRP // NS
GB = 8

_mesh = lambda: plsc.VectorSubcoreMesh(core_axis_name="c", subcore_axis_name="s")
_sc_params = pltpu.CompilerParams(use_tc_tiling_on_sc=False)


def _deg_body(col2d, deg0, deg1, deg_sp, onesbuf, idxbuf, zbuf):
    c = lax.axis_index("c")
    s = lax.axis_index("s")
    lane = lax.iota(jnp.int32, 16)
    onev = jnp.where(lane == 0, 1.0, 0.0).astype(jnp.float32)
    zv = jnp.zeros((16,), jnp.float32)

    @pl.loop(0, 128)
    def _(r):
        onesbuf[r] = onev

    @pl.loop(0, CR)
    def _(r):
        zbuf[r] = zv

    @pl.loop(0, RPT // CR)
    def _(k):
        pltpu.sync_copy(zbuf, deg_sp.at[pl.ds(s * RPT + k * CR, CR)])

    plsc.subcore_barrier()

    base = (c * NS + s) * (ER // 32)

    @pl.loop(0, 5)
    def _(kc):
        pltpu.sync_copy(col2d.at[pl.ds(base + kc * 40, 40)], idxbuf)

        @pl.loop(0, 40)
        def _(j):
            pltpu.sync_copy(onesbuf, deg_sp.at[idxbuf.at[j]], add=True)

    plsc.subcore_barrier()

    @pl.when(c == 0)
    def _():
        @pl.loop(0, RPT // CR)
        def _(k):
            r0 = s * RPT + k * CR
            pltpu.sync_copy(deg_sp.at[pl.ds(r0, CR)], deg0.at[pl.ds(r0, CR)])

    @pl.when(c == 1)
    def _():
        @pl.loop(0, RPT // CR)
        def _(k):
            r0 = s * RPT + k * CR
            pltpu.sync_copy(deg_sp.at[pl.ds(r0, CR)], deg1.at[pl.ds(r0, CR)])


_deg_kernel = pl.kernel(
    _deg_body,
    out_type=[jax.ShapeDtypeStruct((NP, 16), jnp.float32)] * 2,
    mesh=_mesh(),
    compiler_params=_sc_params,
    scratch_types=[
        pltpu.VMEM_SHARED((NP, 16), jnp.float32),
        pltpu.VMEM((128, 16), jnp.float32),
        pltpu.VMEM((40, 128), jnp.int32),
        pltpu.VMEM((CR, 16), jnp.float32),
    ],
)


BN2 = 3136


def _prep_body(dp0, dp1, x0, dis, ylo, yhi):
    i = pl.program_id(0)
    deg = dp0[...][:, 0] + dp1[...][:, 0]
    d = jnp.where(deg > 0, lax.rsqrt(deg), 0.0)
    dis[pl.ds(pl.multiple_of(i * BN2, 128), BN2)] = d
    y = x0[...] * d[:, None]
    ylo[...] = y[:, :HD]
    yhi[...] = y[:, HD:]


_prep = pl.pallas_call(
    _prep_body,
    grid=(NP // BN2,),
    in_specs=[
        pl.BlockSpec((BN2, 16), lambda i: (i, 0)),
        pl.BlockSpec((BN2, 16), lambda i: (i, 0)),
        pl.BlockSpec((BN2, D), lambda i: (i, 0)),
    ],
    out_specs=[
        pl.BlockSpec((NP,), lambda i: (0,)),
        pl.BlockSpec((BN2, HD), lambda i: (i, 0)),
        pl.BlockSpec((BN2, HD), lambda i: (i, 0)),
    ],
    out_shape=[
        jax.ShapeDtypeStruct((NP,), jnp.float32),
        jax.ShapeDtypeStruct((NP, HD), jnp.float32),
        jax.ShapeDtypeStruct((NP, HD), jnp.float32),
    ],
)


def _layer_body(alpha, y_lo, y_hi, row2d, col2d, dis, zin_lo, zin_hi,
                yn_lo, yn_hi, zo_lo, zo_hi,
                acc, rbuf, ridx, cidx, ebuf, zbuf, dbuf, sem):
    c = lax.axis_index("c")
    s = lax.axis_index("s")

    def half(y, zin, yn, zo):
        @pl.loop(0, RPT // CR)
        def _(k):
            r0 = s * RPT + k * CR
            pltpu.sync_copy(y.at[pl.ds(r0, CR)], acc.at[pl.ds(r0, CR)])

        plsc.subcore_barrier()

        base = s * EPTR

        @pl.loop(0, NIT)
        def _(it):
            pltpu.sync_copy(row2d.at[pl.ds(base + it * KB, KB)], ridx)
            pltpu.sync_copy(col2d.at[pl.ds(base + it * KB, KB)], cidx)
            descs = [
                pltpu.async_copy(y.at[ridx.at[j]], rbuf.at[j], sem)
                for j in range(KB)
            ]
            for dsc in descs:
                dsc.wait()
            for j in range(KB):
                pltpu.sync_copy(rbuf.at[j], acc.at[cidx.at[j]], add=True)

        plsc.subcore_barrier()

        @pl.loop(0, RPT // CR)
        def _(k):
            r0 = s * RPT + k * CR
            pltpu.sync_copy(acc.at[pl.ds(r0, CR)], ebuf)
            pltpu.sync_copy(zin.at[pl.ds(r0, CR)], zbuf)
            pltpu.sync_copy(dis.at[pl.ds(r0, CR)], dbuf)

            @pl.loop(0, CR // 16)
            def _(g):
                dvec = dbuf[pl.ds(g * 16, 16)]
                for j in range(16):
                    r = g * 16 + j
                    dv = dvec[j]
                    a0 = ebuf[r, pl.ds(0, 16)]
                    a1 = ebuf[r, pl.ds(16, 16)]
                    x0v = dv * a0
                    x1v = dv * a1
                    zbuf[r, pl.ds(0, 16)] = (zbuf[r, pl.ds(0, 16)] + x0v) * alpha
                    zbuf[r, pl.ds(16, 16)] = (zbuf[r, pl.ds(16, 16)] + x1v) * alpha
                    ebuf[r, pl.ds(0, 16)] = dv * x0v
                    ebuf[r, pl.ds(16, 16)] = dv * x1v

            pltpu.sync_copy(ebuf, yn.at[pl.ds(r0, CR)])
            pltpu.sync_copy(zbuf, zo.at[pl.ds(r0, CR)])

    @pl.when(c == 0)
    def _():
        half(y_lo, zin_lo, yn_lo, zo_lo)

    @pl.when(c == 1)
    def _():
        half(y_hi, zin_hi, yn_hi, zo_hi)


def _make_layer(alpha):
    return pl.kernel(
        functools.partial(_layer_body, alpha),
        out_type=[jax.ShapeDtypeStruct((NP, HD), jnp.float32)] * 4,
        mesh=_mesh(),
    compiler_params=_sc_params,
        scratch_types=[
            pltpu.VMEM_SHARED((NP, HD), jnp.float32),
            pltpu.VMEM((KB, 128, HD), jnp.float32),
            pltpu.VMEM((KB, 128), jnp.int32),
            pltpu.VMEM((KB, 128), jnp.int32),
            pltpu.VMEM((CR, HD), jnp.float32),
            pltpu.VMEM((CR, HD), jnp.float32),
            pltpu.VMEM((CR,), jnp.float32),
            pltpu.SemaphoreType.DMA,
        ],
    )


_layer1 = _make_layer(1.0)
_layer2 = _make_layer(1.0 / 3.0)


def _gather_body(z_lo, z_hi, idx2d, out_lo, out_hi, gidx, gbuf, sem):
    c = lax.axis_index("c")
    s = lax.axis_index("s")

    def half(z, out):
        base = s * GPT
        pltpu.sync_copy(idx2d.at[pl.ds(base, GPT)], gidx)

        @pl.loop(0, GPT // GB)
        def _(it):
            descs = [
                pltpu.async_copy(z.at[gidx.at[it * GB + j]],
                                 gbuf.at[pl.ds(j * 128, 128)], sem)
                for j in range(GB)
            ]
            for dsc in descs:
                dsc.wait()
            pltpu.sync_copy(gbuf, out.at[pl.ds((base + it * GB) * 128, GB * 128)])

    @pl.when(c == 0)
    def _():
        half(z_lo, out_lo)

    @pl.when(c == 1)
    def _():
        half(z_hi, out_hi)


_gather = pl.kernel(
    _gather_body,
    out_type=[jax.ShapeDtypeStruct((GIP, HD), jnp.float32)] * 2,
    mesh=_mesh(),
    compiler_params=_sc_params,
    scratch_types=[
        pltpu.VMEM((GPT, 128), jnp.int32),
        pltpu.VMEM((GB * 128, HD), jnp.float32),
        pltpu.SemaphoreType.DMA,
    ],
)


BS = 512


def _loss_body(ulo, uhi, vlo, vhi, nlo, nhi, w, out):
    i = pl.program_id(0)
    ul = ulo[...]
    uh = uhi[...]
    pos = jnp.sum(ul * vlo[...], axis=1) + jnp.sum(uh * vhi[...], axis=1)
    nl = nlo[...]
    nh = nhi[...]
    neg = jnp.sum(nl * ul[:, None, :], axis=2) + jnp.sum(nh * uh[:, None, :], axis=2)
    coef = -jnp.sign(w[...]) + 2.0
    t = coef[:, None] * pos[:, None] - neg
    ls = jnp.minimum(t, 0.0) - jnp.log1p(jnp.exp(-jnp.abs(t)))
    reg = (jnp.sum(ul * ul) + jnp.sum(uh * uh)
           + jnp.sum(vlo[...] ** 2) + jnp.sum(vhi[...] ** 2)
           + jnp.sum(nl * nl) + jnp.sum(nh * nh))
    val = -jnp.sum(ls) + REG * reg

    @pl.when(i == 0)
    def _():
        out[0, 0] = 0.0

    out[0, 0] += val


_loss = pl.pallas_call(
    _loss_body,
    grid=(B // BS,),
    in_specs=[
        pl.BlockSpec((BS, HD), lambda i: (i, 0)),
        pl.BlockSpec((BS, HD), lambda i: (i, 0)),
        pl.BlockSpec((BS, HD), lambda i: (i, 0)),
        pl.BlockSpec((BS, HD), lambda i: (i, 0)),
        pl.BlockSpec((BS, NNEG, HD), lambda i: (i, 0, 0)),
        pl.BlockSpec((BS, NNEG, HD), lambda i: (i, 0, 0)),
        pl.BlockSpec((BS,), lambda i: (i,)),
    ],
    out_specs=pl.BlockSpec((1, 1), lambda i: (0, 0), memory_space=pltpu.SMEM),
    out_shape=jax.ShapeDtypeStruct((1, 1), jnp.float32),
)


def kernel(E_pos, E_item, u, v, w, n, edge_index):
    x0 = jnp.concatenate([E_pos, E_item], axis=0)
    x0p = jnp.pad(x0, ((0, NP - NN), (0, 0)))
    pad = jnp.full((EP - E,), DUMMY, jnp.int32)
    row2d = jnp.concatenate([edge_index[0], pad]).reshape(ER, 128)
    col2d = jnp.concatenate([edge_index[1], pad]).reshape(ER, 128)

    deg0, deg1 = _deg_kernel(col2d)
    dis, y_lo, y_hi = _prep(deg0, deg1, x0p)

    zin_lo = x0p[:, :HD]
    zin_hi = x0p[:, HD:]
    y1lo, y1hi, z1lo, z1hi = _layer1(y_lo, y_hi, row2d, col2d, dis, zin_lo, zin_hi)
    _y2lo, _y2hi, zlo, zhi = _layer2(y1lo, y1hi, row2d, col2d, dis, z1lo, z1hi)

    gpad = jnp.full((GIP - GI,), DUMMY, jnp.int32)
    all_idx = jnp.concatenate([u, v, n.reshape(-1), gpad]).reshape(GRP, 128)
    rows_lo, rows_hi = _gather(zlo, zhi, all_idx)

    ulo, vlo_r, nlo = rows_lo[:B], rows_lo[B:2 * B], rows_lo[2 * B:GI]
    uhi, vhi_r, nhi = rows_hi[:B], rows_hi[B:2 * B], rows_hi[2 * B:GI]
    out = _loss(ulo, uhi, vlo_r, vhi_r,
                nlo.reshape(B, NNEG, HD), nhi.reshape(B, NNEG, HD), w)
    return out[0, 0]
